# baseline (speedup 1.0000x reference)
import functools

import jax
import jax.numpy as jnp
from jax import lax
from jax.experimental import pallas as pl
from jax.experimental.pallas import tpu as pltpu
from jax.experimental.pallas import tpu_sc as plsc

V = 1000000
D = 64
H = 256
B = 16384
EPS = 1e-5

@functools.cache
def _make_sc_gather():
    info = plsc.get_sparse_core_info()
    nc, ns = info.num_cores, info.num_subcores
    bpw = B // (nc * ns)

    def _gather_body(table_hbm, idx_hbm, out_hbm, idx_v, rows_v, sem):
        wid = lax.axis_index("s") * nc + lax.axis_index("c")
        base = wid * bpw
        pltpu.sync_copy(idx_hbm.at[pl.ds(base, bpw)], idx_v)

        def issue(g, carry):
            vec = idx_v[pl.ds(g * 16, 16)]
            for l in range(16):
                pltpu.async_copy(
                    table_hbm.at[pl.ds(vec[l], 1)],
                    rows_v.at[pl.ds(g * 16 + l, 1)], sem)
            return carry

        lax.fori_loop(0, bpw // 16, issue, 0)
        pltpu.make_async_copy(
            table_hbm.at[pl.ds(0, bpw)], rows_v, sem).wait()
        pltpu.sync_copy(rows_v, out_hbm.at[pl.ds(base, bpw)])

    return pl.kernel(
        _gather_body,
        out_type=jax.ShapeDtypeStruct((B, D), jnp.float32),
        mesh=plsc.VectorSubcoreMesh(core_axis_name="c", subcore_axis_name="s"),
        scratch_types=[
            pltpu.VMEM((bpw,), jnp.int32),
            pltpu.VMEM((bpw, D), jnp.float32),
            pltpu.SemaphoreType.DMA,
        ],
    )


_BLK = 2048
_NB = B // _BLK


def _stats_body(emb_ref, w1_ref, b1_ref, out_ref):
    i = pl.program_id(0)

    @pl.when(i == 0)
    def _():
        out_ref[...] = jnp.zeros_like(out_ref)

    h = jnp.maximum(
        jnp.dot(emb_ref[...], w1_ref[...],
                preferred_element_type=jnp.float32) + b1_ref[...], 0.0)
    out_ref[0:1, :] += jnp.sum(h, axis=0, keepdims=True)
    out_ref[1:2, :] += jnp.sum(h * h, axis=0, keepdims=True)


def _out_body(stats_ref, emb_ref, w1_ref, b1_ref, gamma_ref, beta_ref,
              w2_ref, b2_ref, out_ref):
    mean = stats_ref[0:1, :] * (1.0 / B)
    var = stats_ref[1:2, :] * (1.0 / B) - mean * mean
    scale = gamma_ref[...] * lax.rsqrt(var + EPS)
    w2 = w2_ref[...]
    w2p = w2 * scale.reshape(H, 1)
    bias = jnp.dot(beta_ref[...] - mean * scale, w2,
                   preferred_element_type=jnp.float32) + b2_ref[...]
    h = jnp.maximum(
        jnp.dot(emb_ref[...], w1_ref[...],
                preferred_element_type=jnp.float32) + b1_ref[...], 0.0)
    out_ref[...] = jnp.maximum(
        jnp.dot(h, w2p, preferred_element_type=jnp.float32) + bias, 0.0)


def _mlp(emb, W1, b1, gamma, beta, W2, b2):
    b1r = b1.reshape(1, H)
    const = lambda i: (0, 0)
    stats = pl.pallas_call(
        _stats_body,
        grid=(_NB,),
        in_specs=[
            pl.BlockSpec((_BLK, D), lambda i: (i, 0)),
            pl.BlockSpec((D, H), const),
            pl.BlockSpec((1, H), const),
        ],
        out_specs=pl.BlockSpec((2, H), const),
        out_shape=jax.ShapeDtypeStruct((2, H), jnp.float32),
        compiler_params=pltpu.CompilerParams(
            dimension_semantics=("arbitrary",)),
    )(emb, W1, b1r)
    return pl.pallas_call(
        _out_body,
        grid=(_NB,),
        in_specs=[
            pl.BlockSpec((2, H), const),
            pl.BlockSpec((_BLK, D), lambda i: (i, 0)),
            pl.BlockSpec((D, H), const),
            pl.BlockSpec((1, H), const),
            pl.BlockSpec((1, H), const),
            pl.BlockSpec((1, H), const),
            pl.BlockSpec((H, H), const),
            pl.BlockSpec((1, H), const),
        ],
        out_specs=pl.BlockSpec((_BLK, H), lambda i: (i, 0)),
        out_shape=jax.ShapeDtypeStruct((B, H), jnp.float32),
        compiler_params=pltpu.CompilerParams(
            dimension_semantics=("arbitrary",)),
    )(stats, emb, W1, b1r, gamma.reshape(1, H), beta.reshape(1, H), W2,
      b2.reshape(1, H))


def kernel(target_ids, table, W1, b1, gamma, beta, W2, b2):
    emb = _make_sc_gather()(table, target_ids.astype(jnp.int32))
    return _mlp(emb, W1, b1, gamma, beta, W2, b2)


# speedup vs baseline: 1.1663x; 1.1663x over previous
import functools

import jax
import jax.numpy as jnp
from jax import lax
from jax.experimental import pallas as pl
from jax.experimental.pallas import tpu as pltpu
from jax.experimental.pallas import tpu_sc as plsc

V = 1000000
D = 64
H = 256
B = 16384
EPS = 1e-5

_NBLK = 7812
_TAILCUT = _NBLK * 128
_TAIL0 = V - 128


def _scalar(x):
    return x if x.ndim == 0 else x[0]


@functools.cache
def _make_sc_gather():
    info = plsc.get_sparse_core_info()
    nc, ns = info.num_cores, info.num_subcores
    nw = nc * ns
    blk_per_w = -(-_NBLK // nw)

    def body(tableT_hbm, ttail_hbm, idx_hbm, out_hbm,
             ids_v, myids_v, mypos_v, buf_v, strips_v, bsem, wsem, tsem):
        wid = lax.axis_index("s") * nc + lax.axis_index("c")
        lo = wid * blk_per_w
        hi = jnp.minimum(lo + blk_per_w, _NBLK)
        nb = hi - lo
        iota = lax.broadcasted_iota(jnp.int32, (16,), 0)
        pltpu.sync_copy(idx_hbm, ids_v)

        def scan(g, cnt):
            v = ids_v[pl.ds(g * 16, 16)]
            cb = v >> 7
            m = (cb >= lo) & (cb < hi) & (v < _TAILCUT)
            c = plsc.cumsum(m.astype(jnp.int32))
            pos = cnt + c - 1
            plsc.store_scatter(myids_v, [pos], v, mask=m)
            plsc.store_scatter(mypos_v, [pos], g * 16 + iota, mask=m)
            return cnt + c[15]

        cnt = lax.fori_loop(0, B // 16, scan, jnp.int32(0))
        ngrp = (cnt + 15) >> 4

        def issue_blk(k, slot):
            off = pl.multiple_of((lo + k) * 128, 128)
            pltpu.async_copy(tableT_hbm.at[:, pl.ds(off, 128)],
                             buf_v.at[slot], bsem.at[slot])

        issue_blk(0, 0)

        def extract_one(mv, pv, sel, wc, cb, slot):
            l0 = plsc.all_reduce_ffs(sel)
            l0v = jnp.broadcast_to(_scalar(l0), (16,))
            idv = mv.at[l0v].get(mode="promise_in_bounds")
            posv = pv.at[l0v].get(mode="promise_in_bounds")
            lane = idv - cb * 128

            @pl.when(wc >= 16)
            def _():
                pltpu.make_async_copy(out_hbm.at[pl.ds(0, 1)],
                                      strips_v.at[pl.ds(0, 1)], wsem).wait()

            s = lax.rem(wc, jnp.int32(16))
            slotv = jnp.broadcast_to(slot, (16,))
            for q in range(4):
                col = plsc.load_gather(
                    buf_v, [slotv, iota + q * 16, lane])
                strips_v[s, pl.ds(q * 16, 16)] = col
            pltpu.async_copy(strips_v.at[pl.ds(s, 1)],
                             out_hbm.at[pl.ds(_scalar(posv), 1)], wsem)
            return sel & (iota != l0v), wc + 1

        def blk_loop(k, wcount):
            slot = lax.rem(k, jnp.int32(2))

            @pl.when(k + 1 < nb)
            def _():
                issue_blk(k + 1, lax.rem(k + 1, jnp.int32(2)))

            pltpu.make_async_copy(tableT_hbm.at[:, pl.ds(0, 128)],
                                  buf_v.at[slot], bsem.at[slot]).wait()
            cb = lo + k

            def grp(g2, wc):
                mv = myids_v[pl.ds(g2 * 16, 16)]
                pv = mypos_v[pl.ds(g2 * 16, 16)]
                valid = (g2 * 16 + iota) < cnt
                sel0 = ((mv >> 7) == cb) & valid
                rem0 = _scalar(plsc.all_reduce_population_count(sel0))

                def cond(st):
                    return st[2] > 0

                def step(st):
                    sel2, wc3 = extract_one(mv, pv, st[0], st[1], cb, slot)
                    return sel2, wc3, st[2] - 1

                _, wc2, _ = lax.while_loop(cond, step, (sel0, wc, rem0))
                return wc2

            return lax.fori_loop(0, ngrp, grp, wcount)

        wcount = lax.fori_loop(0, nb, blk_loop, jnp.int32(0))

        def drain(_, carry):
            pltpu.make_async_copy(out_hbm.at[pl.ds(0, 1)],
                                  strips_v.at[pl.ds(0, 1)], wsem).wait()
            return carry

        lax.fori_loop(0, jnp.minimum(wcount, 16), drain, 0)

        @pl.when(wid == 0)
        def _():
            def tscan(g, carry):
                v = ids_v[pl.ds(g * 16, 16)]
                sel_init = v >= _TAILCUT
                trem = _scalar(plsc.all_reduce_population_count(sel_init))

                def cond(st):
                    return st[1] > 0

                def step(st):
                    sel, c2 = st
                    l0 = plsc.all_reduce_ffs(sel)
                    l0v = jnp.broadcast_to(_scalar(l0), (16,))
                    idv = v.at[l0v].get(mode="promise_in_bounds")
                    row = _scalar(idv) - _TAIL0
                    pos = g * 16 + _scalar(l0)
                    pltpu.async_copy(ttail_hbm.at[pl.ds(row, 1)],
                                     strips_v.at[pl.ds(0, 1)], tsem)
                    pltpu.make_async_copy(
                        ttail_hbm.at[pl.ds(0, 1)],
                        strips_v.at[pl.ds(0, 1)], tsem).wait()
                    pltpu.async_copy(strips_v.at[pl.ds(0, 1)],
                                     out_hbm.at[pl.ds(pos, 1)], tsem)
                    pltpu.make_async_copy(
                        ttail_hbm.at[pl.ds(0, 1)],
                        strips_v.at[pl.ds(0, 1)], tsem).wait()
                    return sel & (iota != l0v), c2 - 1

                lax.while_loop(cond, step, (sel_init, trem))
                return carry

            lax.fori_loop(0, B // 16, tscan, 0)

    return pl.kernel(
        body,
        out_type=jax.ShapeDtypeStruct((B, D), jnp.float32),
        mesh=plsc.VectorSubcoreMesh(core_axis_name="c", subcore_axis_name="s"),
        scratch_types=[
            pltpu.VMEM((B,), jnp.int32),
            pltpu.VMEM((B,), jnp.int32),
            pltpu.VMEM((B,), jnp.int32),
            pltpu.VMEM((2, D, 128), jnp.float32),
            pltpu.VMEM((16, D), jnp.float32),
            pltpu.SemaphoreType.DMA((2,)),
            pltpu.SemaphoreType.DMA,
            pltpu.SemaphoreType.DMA,
        ],
        compiler_params=pltpu.CompilerParams(needs_layout_passes=False),
    )


_BLK = 2048
_NB = B // _BLK


def _h_block(emb_blk, w1, b1):
    return jnp.maximum(
        jnp.dot(emb_blk, w1, preferred_element_type=jnp.float32) + b1, 0.0)


def _stats_body(emb_ref, w1_ref, b1_ref, out_ref):
    i = pl.program_id(0)

    @pl.when(i == 0)
    def _():
        out_ref[...] = jnp.zeros_like(out_ref)

    h = _h_block(emb_ref[...], w1_ref[...], b1_ref[...])
    out_ref[0:1, :] += jnp.sum(h, axis=0, keepdims=True)
    out_ref[1:2, :] += jnp.sum(h * h, axis=0, keepdims=True)


def _out_body(stats_ref, emb_ref, w1_ref, b1_ref, gamma_ref, beta_ref,
              w2_ref, b2_ref, out_ref):
    mean = stats_ref[0:1, :] * (1.0 / B)
    var = stats_ref[1:2, :] * (1.0 / B) - mean * mean
    scale = gamma_ref[...] * lax.rsqrt(var + EPS)
    w2 = w2_ref[...]
    w2p = w2 * scale.reshape(H, 1)
    bias = jnp.dot(beta_ref[...] - mean * scale, w2,
                   preferred_element_type=jnp.float32) + b2_ref[...]
    h = _h_block(emb_ref[...], w1_ref[...], b1_ref[...])
    out_ref[...] = jnp.maximum(
        jnp.dot(h, w2p, preferred_element_type=jnp.float32) + bias, 0.0)


def _mlp(emb, W1, b1, gamma, beta, W2, b2):
    b1r = b1.reshape(1, H)
    const = lambda i: (0, 0)
    stats = pl.pallas_call(
        _stats_body,
        grid=(_NB,),
        in_specs=[
            pl.BlockSpec((_BLK, D), lambda i: (i, 0)),
            pl.BlockSpec((D, H), const),
            pl.BlockSpec((1, H), const),
        ],
        out_specs=pl.BlockSpec((2, H), const),
        out_shape=jax.ShapeDtypeStruct((2, H), jnp.float32),
        compiler_params=pltpu.CompilerParams(
            dimension_semantics=("arbitrary",)),
    )(emb, W1, b1r)
    return pl.pallas_call(
        _out_body,
        grid=(_NB,),
        in_specs=[
            pl.BlockSpec((2, H), const),
            pl.BlockSpec((_BLK, D), lambda i: (i, 0)),
            pl.BlockSpec((D, H), const),
            pl.BlockSpec((1, H), const),
            pl.BlockSpec((1, H), const),
            pl.BlockSpec((1, H), const),
            pl.BlockSpec((H, H), const),
            pl.BlockSpec((1, H), const),
        ],
        out_specs=pl.BlockSpec((_BLK, H), lambda i: (i, 0)),
        out_shape=jax.ShapeDtypeStruct((B, H), jnp.float32),
        compiler_params=pltpu.CompilerParams(
            dimension_semantics=("arbitrary",)),
    )(stats, emb, W1, b1r, gamma.reshape(1, H), beta.reshape(1, H), W2,
      b2.reshape(1, H))


def kernel(target_ids, table, W1, b1, gamma, beta, W2, b2):
    tableT = table.T
    ttail = lax.slice(table, (_TAIL0, 0), (V, D))
    emb = _make_sc_gather()(tableT, ttail, target_ids.astype(jnp.int32))
    return _mlp(emb, W1, b1, gamma, beta, W2, b2)


# speedup vs baseline: 1.6464x; 1.4117x over previous
import functools

import jax
import jax.numpy as jnp
from jax import lax
from jax.experimental import pallas as pl
from jax.experimental.pallas import tpu as pltpu
from jax.experimental.pallas import tpu_sc as plsc

V = 1000000
D = 64
H = 256
B = 16384
EPS = 1e-5

_BW = 256
_NBLK = 3906
_TAILCUT = _NBLK * _BW
_TAIL0 = V - 128


def _scalar(x):
    return x if x.ndim == 0 else x[0]


@functools.cache
def _make_sc_gather():
    info = plsc.get_sparse_core_info()
    nc, ns = info.num_cores, info.num_subcores
    nw = nc * ns
    blk_per_w = -(-_NBLK // nw)

    def body(tableT_hbm, ttail_hbm, idx_hbm, out_hbm,
             ids_v, myblk_v, myids_v, mypos_v, buf_v, strips_v,
             bsem, wsem, tsem):
        wid = lax.axis_index("s") * nc + lax.axis_index("c")
        lo = wid * blk_per_w
        hi = jnp.minimum(lo + blk_per_w, _NBLK)
        nb = hi - lo
        iota = lax.broadcasted_iota(jnp.int32, (16,), 0)
        pltpu.sync_copy(idx_hbm, ids_v)

        def scan(g, cnt):
            v = ids_v[pl.ds(g * 16, 16)]
            cb = v >> 8
            m = (cb >= lo) & (cb < hi) & (v < _TAILCUT)
            c = plsc.cumsum(m.astype(jnp.int32))
            pos = cnt + c - 1
            plsc.store_scatter(myblk_v, [pos], cb, mask=m)
            plsc.store_scatter(myids_v, [pos], v, mask=m)
            plsc.store_scatter(mypos_v, [pos], g * 16 + iota, mask=m)
            return cnt + c[15]

        cnt = lax.fori_loop(0, B // 16, scan, jnp.int32(0))
        plsc.store_scatter(myblk_v, [cnt + iota],
                           jnp.broadcast_to(jnp.int32(-1), (16,)))
        ngrp = (cnt + 15) >> 4

        def issue_blk(k, slot):
            off = pl.multiple_of((lo + k) * _BW, _BW)
            pltpu.async_copy(tableT_hbm.at[:, pl.ds(off, _BW)],
                             buf_v.at[slot], bsem.at[slot])

        issue_blk(0, 0)

        def extract_one(g2, sel, wc, cb, slot):
            l0 = plsc.all_reduce_ffs(sel)
            l0v = jnp.broadcast_to(_scalar(l0), (16,))
            gidx = g2 * 16 + l0v
            idv = plsc.load_gather(myids_v, [gidx])
            posv = plsc.load_gather(mypos_v, [gidx])
            lane = idv - cb * _BW

            @pl.when(wc >= 16)
            def _():
                pltpu.make_async_copy(out_hbm.at[pl.ds(0, 1)],
                                      strips_v.at[pl.ds(0, 1)], wsem).wait()

            s = lax.rem(wc, jnp.int32(16))
            slotv = jnp.broadcast_to(slot, (16,))
            for q in range(4):
                col = plsc.load_gather(
                    buf_v, [slotv, iota + q * 16, lane])
                strips_v[s, pl.ds(q * 16, 16)] = col
            pltpu.async_copy(strips_v.at[pl.ds(s, 1)],
                             out_hbm.at[pl.ds(_scalar(posv), 1)], wsem)
            return sel & (iota != l0v), wc + 1

        def blk_loop(k, wcount):
            slot = lax.rem(k, jnp.int32(2))

            @pl.when(k + 1 < nb)
            def _():
                issue_blk(k + 1, lax.rem(k + 1, jnp.int32(2)))

            pltpu.make_async_copy(tableT_hbm.at[:, pl.ds(0, _BW)],
                                  buf_v.at[slot], bsem.at[slot]).wait()
            cb = lo + k

            def grp(g2, wc):
                sel0 = myblk_v[pl.ds(g2 * 16, 16)] == cb
                rem0 = _scalar(plsc.all_reduce_population_count(sel0))

                def cond(st):
                    return st[2] > 0

                def step(st):
                    sel2, wc3 = extract_one(g2, st[0], st[1], cb, slot)
                    return sel2, wc3, st[2] - 1

                _, wc2, _ = lax.while_loop(cond, step, (sel0, wc, rem0))
                return wc2

            return lax.fori_loop(0, ngrp, grp, wcount)

        wcount = lax.fori_loop(0, nb, blk_loop, jnp.int32(0))

        def drain(_, carry):
            pltpu.make_async_copy(out_hbm.at[pl.ds(0, 1)],
                                  strips_v.at[pl.ds(0, 1)], wsem).wait()
            return carry

        lax.fori_loop(0, jnp.minimum(wcount, 16), drain, 0)

        @pl.when(wid == 0)
        def _():
            def tscan(g, carry):
                v = ids_v[pl.ds(g * 16, 16)]
                sel_init = v >= _TAILCUT
                trem = _scalar(plsc.all_reduce_population_count(sel_init))

                def cond(st):
                    return st[1] > 0

                def step(st):
                    sel, c2 = st
                    l0 = plsc.all_reduce_ffs(sel)
                    l0v = jnp.broadcast_to(_scalar(l0), (16,))
                    idv = v.at[l0v].get(mode="promise_in_bounds")
                    row = _scalar(idv) - _TAIL0
                    pos = g * 16 + _scalar(l0)
                    pltpu.async_copy(ttail_hbm.at[pl.ds(row, 1)],
                                     strips_v.at[pl.ds(0, 1)], tsem)
                    pltpu.make_async_copy(
                        ttail_hbm.at[pl.ds(0, 1)],
                        strips_v.at[pl.ds(0, 1)], tsem).wait()
                    pltpu.async_copy(strips_v.at[pl.ds(0, 1)],
                                     out_hbm.at[pl.ds(pos, 1)], tsem)
                    pltpu.make_async_copy(
                        ttail_hbm.at[pl.ds(0, 1)],
                        strips_v.at[pl.ds(0, 1)], tsem).wait()
                    return sel & (iota != l0v), c2 - 1

                lax.while_loop(cond, step, (sel_init, trem))
                return carry

            lax.fori_loop(0, B // 16, tscan, 0)

    return pl.kernel(
        body,
        out_type=jax.ShapeDtypeStruct((B, D), jnp.float32),
        mesh=plsc.VectorSubcoreMesh(core_axis_name="c", subcore_axis_name="s"),
        scratch_types=[
            pltpu.VMEM((B,), jnp.int32),
            pltpu.VMEM((B + 16,), jnp.int32),
            pltpu.VMEM((B,), jnp.int32),
            pltpu.VMEM((B,), jnp.int32),
            pltpu.VMEM((2, D, _BW), jnp.float32),
            pltpu.VMEM((16, D), jnp.float32),
            pltpu.SemaphoreType.DMA((2,)),
            pltpu.SemaphoreType.DMA,
            pltpu.SemaphoreType.DMA,
        ],
        compiler_params=pltpu.CompilerParams(needs_layout_passes=False),
    )


_BLK = 2048
_NB = B // _BLK


def _h_block(emb_blk, w1, b1):
    return jnp.maximum(
        jnp.dot(emb_blk, w1, preferred_element_type=jnp.float32) + b1, 0.0)


def _stats_body(emb_ref, w1_ref, b1_ref, out_ref):
    i = pl.program_id(0)

    @pl.when(i == 0)
    def _():
        out_ref[...] = jnp.zeros_like(out_ref)

    h = _h_block(emb_ref[...], w1_ref[...], b1_ref[...])
    out_ref[0:1, :] += jnp.sum(h, axis=0, keepdims=True)
    out_ref[1:2, :] += jnp.sum(h * h, axis=0, keepdims=True)


def _out_body(stats_ref, emb_ref, w1_ref, b1_ref, gamma_ref, beta_ref,
              w2_ref, b2_ref, out_ref):
    mean = stats_ref[0:1, :] * (1.0 / B)
    var = stats_ref[1:2, :] * (1.0 / B) - mean * mean
    scale = gamma_ref[...] * lax.rsqrt(var + EPS)
    w2 = w2_ref[...]
    w2p = w2 * scale.reshape(H, 1)
    bias = jnp.dot(beta_ref[...] - mean * scale, w2,
                   preferred_element_type=jnp.float32) + b2_ref[...]
    h = _h_block(emb_ref[...], w1_ref[...], b1_ref[...])
    out_ref[...] = jnp.maximum(
        jnp.dot(h, w2p, preferred_element_type=jnp.float32) + bias, 0.0)


def _mlp(emb, W1, b1, gamma, beta, W2, b2):
    b1r = b1.reshape(1, H)
    const = lambda i: (0, 0)
    stats = pl.pallas_call(
        _stats_body,
        grid=(_NB,),
        in_specs=[
            pl.BlockSpec((_BLK, D), lambda i: (i, 0)),
            pl.BlockSpec((D, H), const),
            pl.BlockSpec((1, H), const),
        ],
        out_specs=pl.BlockSpec((2, H), const),
        out_shape=jax.ShapeDtypeStruct((2, H), jnp.float32),
        compiler_params=pltpu.CompilerParams(
            dimension_semantics=("arbitrary",)),
    )(emb, W1, b1r)
    return pl.pallas_call(
        _out_body,
        grid=(_NB,),
        in_specs=[
            pl.BlockSpec((2, H), const),
            pl.BlockSpec((_BLK, D), lambda i: (i, 0)),
            pl.BlockSpec((D, H), const),
            pl.BlockSpec((1, H), const),
            pl.BlockSpec((1, H), const),
            pl.BlockSpec((1, H), const),
            pl.BlockSpec((H, H), const),
            pl.BlockSpec((1, H), const),
        ],
        out_specs=pl.BlockSpec((_BLK, H), lambda i: (i, 0)),
        out_shape=jax.ShapeDtypeStruct((B, H), jnp.float32),
        compiler_params=pltpu.CompilerParams(
            dimension_semantics=("arbitrary",)),
    )(stats, emb, W1, b1r, gamma.reshape(1, H), beta.reshape(1, H), W2,
      b2.reshape(1, H))


def kernel(target_ids, table, W1, b1, gamma, beta, W2, b2):
    tableT = table.T
    ttail = lax.slice(table, (_TAIL0, 0), (V, D))
    emb = _make_sc_gather()(tableT, ttail, target_ids.astype(jnp.int32))
    return _mlp(emb, W1, b1, gamma, beta, W2, b2)


# speedup vs baseline: 2.0168x; 1.2250x over previous
import functools

import jax
import jax.numpy as jnp
from jax import lax
from jax.experimental import pallas as pl
from jax.experimental.pallas import tpu as pltpu
from jax.experimental.pallas import tpu_sc as plsc

V = 1000000
D = 64
H = 256
B = 16384
EPS = 1e-5

_BW = 512
_NBLK = 1953
_TAILCUT = _NBLK * _BW
_TAIL0 = V - 128


def _scalar(x):
    return x if x.ndim == 0 else x[0]


@functools.cache
def _make_sc_gather():
    info = plsc.get_sparse_core_info()
    nc, ns = info.num_cores, info.num_subcores
    nw = nc * ns
    blk_per_w = -(-_NBLK // nw)

    def body(tableT_hbm, ttail_hbm, idx_hbm, out_hbm,
             ids_v, myblk_v, mypacked_v, buf_v, strips_v,
             bsem, wsem, tsem):
        wid = lax.axis_index("s") * nc + lax.axis_index("c")
        lo = wid * blk_per_w
        hi = jnp.minimum(lo + blk_per_w, _NBLK)
        nb = hi - lo
        iota = lax.broadcasted_iota(jnp.int32, (16,), 0)
        pltpu.sync_copy(idx_hbm, ids_v)

        def scan(g, cnt):
            v = ids_v[pl.ds(g * 16, 16)]
            cb = v >> 9
            m = (cb >= lo) & (cb < hi) & (v < _TAILCUT)
            c = plsc.cumsum(m.astype(jnp.int32))
            pos = cnt + c - 1
            packed = ((g * 16 + iota) << 9) | (v & (_BW - 1))
            plsc.store_scatter(myblk_v, [pos], cb, mask=m)
            plsc.store_scatter(mypacked_v, [pos], packed, mask=m)
            return cnt + c[15]

        cnt = lax.fori_loop(0, B // 16, scan, jnp.int32(0))
        plsc.store_scatter(myblk_v, [cnt + iota],
                           jnp.broadcast_to(jnp.int32(-1), (16,)))
        ngrp = (cnt + 15) >> 4

        def issue_blk(k, slot):
            off = pl.multiple_of((lo + k) * _BW, _BW)
            pltpu.async_copy(tableT_hbm.at[:, pl.ds(off, _BW)],
                             buf_v.at[slot], bsem.at[slot])

        issue_blk(0, 0)

        def extract_one(g2, sel, wc, cb, slot):
            l0 = plsc.all_reduce_ffs(sel)
            l0v = jnp.broadcast_to(_scalar(l0), (16,))
            gidx = g2 * 16 + l0v
            packedv = plsc.load_gather(mypacked_v, [gidx])
            lane = packedv & (_BW - 1)
            posv = packedv >> 9

            @pl.when(wc >= 16)
            def _():
                pltpu.make_async_copy(out_hbm.at[pl.ds(0, 1)],
                                      strips_v.at[pl.ds(0, 1)], wsem).wait()

            s = lax.rem(wc, jnp.int32(16))
            slotv = jnp.broadcast_to(slot, (16,))
            for q in range(4):
                col = plsc.load_gather(
                    buf_v, [slotv, iota + q * 16, lane])
                strips_v[s, pl.ds(q * 16, 16)] = col
            pltpu.async_copy(strips_v.at[pl.ds(s, 1)],
                             out_hbm.at[pl.ds(_scalar(posv), 1)], wsem)
            return sel & (iota != l0v), wc + 1

        def blk_loop(k, wcount):
            slot = lax.rem(k, jnp.int32(2))

            @pl.when(k + 1 < nb)
            def _():
                issue_blk(k + 1, lax.rem(k + 1, jnp.int32(2)))

            pltpu.make_async_copy(tableT_hbm.at[:, pl.ds(0, _BW)],
                                  buf_v.at[slot], bsem.at[slot]).wait()
            cb = lo + k

            def grp(g2, wc):
                sel0 = myblk_v[pl.ds(g2 * 16, 16)] == cb
                rem0 = _scalar(plsc.all_reduce_population_count(sel0))

                def cond(st):
                    return st[2] > 0

                def step(st):
                    sel2, wc3 = extract_one(g2, st[0], st[1], cb, slot)
                    return sel2, wc3, st[2] - 1

                _, wc2, _ = lax.while_loop(cond, step, (sel0, wc, rem0))
                return wc2

            return lax.fori_loop(0, ngrp, grp, wcount)

        wcount = lax.fori_loop(0, nb, blk_loop, jnp.int32(0))

        def drain(_, carry):
            pltpu.make_async_copy(out_hbm.at[pl.ds(0, 1)],
                                  strips_v.at[pl.ds(0, 1)], wsem).wait()
            return carry

        lax.fori_loop(0, jnp.minimum(wcount, 16), drain, 0)

        @pl.when(wid == 0)
        def _():
            def tscan(g, carry):
                v = ids_v[pl.ds(g * 16, 16)]
                sel_init = v >= _TAILCUT
                trem = _scalar(plsc.all_reduce_population_count(sel_init))

                def cond(st):
                    return st[1] > 0

                def step(st):
                    sel, c2 = st
                    l0 = plsc.all_reduce_ffs(sel)
                    l0v = jnp.broadcast_to(_scalar(l0), (16,))
                    idv = v.at[l0v].get(mode="promise_in_bounds")
                    row = _scalar(idv) - _TAIL0
                    pos = g * 16 + _scalar(l0)
                    pltpu.async_copy(ttail_hbm.at[pl.ds(row, 1)],
                                     strips_v.at[pl.ds(0, 1)], tsem)
                    pltpu.make_async_copy(
                        ttail_hbm.at[pl.ds(0, 1)],
                        strips_v.at[pl.ds(0, 1)], tsem).wait()
                    pltpu.async_copy(strips_v.at[pl.ds(0, 1)],
                                     out_hbm.at[pl.ds(pos, 1)], tsem)
                    pltpu.make_async_copy(
                        ttail_hbm.at[pl.ds(0, 1)],
                        strips_v.at[pl.ds(0, 1)], tsem).wait()
                    return sel & (iota != l0v), c2 - 1

                lax.while_loop(cond, step, (sel_init, trem))
                return carry

            lax.fori_loop(0, B // 16, tscan, 0)

    return pl.kernel(
        body,
        out_type=jax.ShapeDtypeStruct((B, D), jnp.float32),
        mesh=plsc.VectorSubcoreMesh(core_axis_name="c", subcore_axis_name="s"),
        scratch_types=[
            pltpu.VMEM((B,), jnp.int32),
            pltpu.VMEM((B + 16,), jnp.int32),
            pltpu.VMEM((B,), jnp.int32),
            pltpu.VMEM((2, D, _BW), jnp.float32),
            pltpu.VMEM((16, D), jnp.float32),
            pltpu.SemaphoreType.DMA((2,)),
            pltpu.SemaphoreType.DMA,
            pltpu.SemaphoreType.DMA,
        ],
        compiler_params=pltpu.CompilerParams(needs_layout_passes=False),
    )


_BLK = 2048
_NB = B // _BLK


def _h_block(emb_blk, w1, b1):
    return jnp.maximum(
        jnp.dot(emb_blk, w1, preferred_element_type=jnp.float32) + b1, 0.0)


def _stats_body(emb_ref, w1_ref, b1_ref, out_ref):
    i = pl.program_id(0)

    @pl.when(i == 0)
    def _():
        out_ref[...] = jnp.zeros_like(out_ref)

    h = _h_block(emb_ref[...], w1_ref[...], b1_ref[...])
    out_ref[0:1, :] += jnp.sum(h, axis=0, keepdims=True)
    out_ref[1:2, :] += jnp.sum(h * h, axis=0, keepdims=True)


def _out_body(stats_ref, emb_ref, w1_ref, b1_ref, gamma_ref, beta_ref,
              w2_ref, b2_ref, out_ref):
    mean = stats_ref[0:1, :] * (1.0 / B)
    var = stats_ref[1:2, :] * (1.0 / B) - mean * mean
    scale = gamma_ref[...] * lax.rsqrt(var + EPS)
    w2 = w2_ref[...]
    w2p = w2 * scale.reshape(H, 1)
    bias = jnp.dot(beta_ref[...] - mean * scale, w2,
                   preferred_element_type=jnp.float32) + b2_ref[...]
    h = _h_block(emb_ref[...], w1_ref[...], b1_ref[...])
    out_ref[...] = jnp.maximum(
        jnp.dot(h, w2p, preferred_element_type=jnp.float32) + bias, 0.0)


def _mlp(emb, W1, b1, gamma, beta, W2, b2):
    b1r = b1.reshape(1, H)
    const = lambda i: (0, 0)
    stats = pl.pallas_call(
        _stats_body,
        grid=(_NB,),
        in_specs=[
            pl.BlockSpec((_BLK, D), lambda i: (i, 0)),
            pl.BlockSpec((D, H), const),
            pl.BlockSpec((1, H), const),
        ],
        out_specs=pl.BlockSpec((2, H), const),
        out_shape=jax.ShapeDtypeStruct((2, H), jnp.float32),
        compiler_params=pltpu.CompilerParams(
            dimension_semantics=("arbitrary",)),
    )(emb, W1, b1r)
    return pl.pallas_call(
        _out_body,
        grid=(_NB,),
        in_specs=[
            pl.BlockSpec((2, H), const),
            pl.BlockSpec((_BLK, D), lambda i: (i, 0)),
            pl.BlockSpec((D, H), const),
            pl.BlockSpec((1, H), const),
            pl.BlockSpec((1, H), const),
            pl.BlockSpec((1, H), const),
            pl.BlockSpec((H, H), const),
            pl.BlockSpec((1, H), const),
        ],
        out_specs=pl.BlockSpec((_BLK, H), lambda i: (i, 0)),
        out_shape=jax.ShapeDtypeStruct((B, H), jnp.float32),
        compiler_params=pltpu.CompilerParams(
            dimension_semantics=("arbitrary",)),
    )(stats, emb, W1, b1r, gamma.reshape(1, H), beta.reshape(1, H), W2,
      b2.reshape(1, H))


def kernel(target_ids, table, W1, b1, gamma, beta, W2, b2):
    tableT = table.T
    ttail = lax.slice(table, (_TAIL0, 0), (V, D))
    emb = _make_sc_gather()(tableT, ttail, target_ids.astype(jnp.int32))
    return _mlp(emb, W1, b1, gamma, beta, W2, b2)
